# Initial kernel scaffold; baseline (speedup 1.0000x reference)
import jax
import jax.numpy as jnp
from jax.experimental import pallas as pl


def kernel(x, pos_emb):
    raise NotImplementedError("write your pallas kernel here")


# speedup vs baseline: 3.0006x; 3.0006x over previous
import jax
import jax.numpy as jnp
from jax.experimental import pallas as pl


def _copy_block(in_ref, out_ref):
    out_ref[...] = in_ref[...]


def kernel(x, pos_emb):
    seq_len = x.shape[1]
    d_model = pos_emb.shape[1]
    block_rows = 1024
    num_blocks = pl.cdiv(seq_len, block_rows)
    return pl.pallas_call(
        _copy_block,
        grid=(num_blocks,),
        in_specs=[pl.BlockSpec((block_rows, d_model), lambda i: (i, 0))],
        out_specs=pl.BlockSpec((block_rows, d_model), lambda i: (i, 0)),
        out_shape=jax.ShapeDtypeStruct((seq_len, d_model), pos_emb.dtype),
    )(pos_emb)


# speedup vs baseline: 3.2591x; 1.0861x over previous
import jax
import jax.numpy as jnp
from jax.experimental import pallas as pl


def _copy_block(in_ref, out_ref):
    out_ref[...] = in_ref[...]


def kernel(x, pos_emb):
    seq_len = x.shape[1]
    d_model = pos_emb.shape[1]
    block_rows = 2048
    num_blocks = pl.cdiv(seq_len, block_rows)
    return pl.pallas_call(
        _copy_block,
        grid=(num_blocks,),
        in_specs=[pl.BlockSpec((block_rows, d_model), lambda i: (i, 0))],
        out_specs=pl.BlockSpec((block_rows, d_model), lambda i: (i, 0)),
        out_shape=jax.ShapeDtypeStruct((seq_len, d_model), pos_emb.dtype),
    )(pos_emb)
